# baseline (speedup 1.0000x reference)
import functools

import jax
import jax.numpy as jnp
from jax import lax
from jax.experimental import pallas as pl

N_E = 1024
E_DIM = 256
BETA = 0.25
B = 16
HW = 1024
T = 1024
NT = HW // T


def _vq_block(z_ref, cb_ref, zq_ref, idx_ref, part_ref):
    zb = z_ref[0]
    cb = cb_ref[...]

    zn = jnp.sum(zb * zb, axis=0, keepdims=True)
    cbn = jnp.sum(cb * cb, axis=1)
    mmT = lax.dot_general(cb, zb, (((1,), (0,)), ((), ())),
                          preferred_element_type=jnp.float32)
    d = zn + cbn[:, None] - 2.0 * mmT

    minval = jnp.min(d, axis=0, keepdims=True)
    iota_f = lax.broadcasted_iota(jnp.int32, (N_E, T), 0).astype(jnp.float32)
    idxf = jnp.min(jnp.where(d == minval, iota_f, float(N_E)), axis=0,
                   keepdims=True)
    idx_ref[0, 0] = idxf[0, :].astype(jnp.int32)

    onehot = jnp.where(iota_f == idxf, 1.0, 0.0).astype(jnp.bfloat16)
    zq = lax.dot_general(cb.astype(jnp.bfloat16), onehot,
                         (((0,), (0,)), ((), ())),
                         preferred_element_type=jnp.float32)
    zq_ref[0] = zq

    diff = zq - zb
    part = jnp.sum(diff * diff)
    part_ref[0, 0] = jnp.broadcast_to(part, (128,))


@functools.partial(jax.jit, static_argnames=())
def kernel(z, codebook):
    z3 = z.reshape(B, E_DIM, HW)
    grid = (B * NT,)

    zq3, idx2, parts = pl.pallas_call(
        _vq_block,
        grid=grid,
        in_specs=[
            pl.BlockSpec((1, E_DIM, T), lambda i: (i // NT, 0, i % NT)),
            pl.BlockSpec((N_E, E_DIM), lambda i: (0, 0)),
        ],
        out_specs=[
            pl.BlockSpec((1, E_DIM, T), lambda i: (i // NT, 0, i % NT)),
            pl.BlockSpec((1, 1, T), lambda i: (i, 0, 0)),
            pl.BlockSpec((1, 1, 128), lambda i: (i, 0, 0)),
        ],
        out_shape=[
            jax.ShapeDtypeStruct((B, E_DIM, HW), jnp.float32),
            jax.ShapeDtypeStruct((B * NT, 1, T), jnp.int32),
            jax.ShapeDtypeStruct((B * NT, 1, 128), jnp.float32),
        ],
    )(z3, codebook)

    z_q_out = zq3.reshape(z.shape)
    loss = (1.0 + BETA) * jnp.sum(parts[:, 0, 0]) / (B * HW * E_DIM)
    indices_out = idx2.reshape(B, 1, 32, 32)
    return (z_q_out, loss, indices_out)


# speedup vs baseline: 1.0019x; 1.0019x over previous
import functools

import jax
import jax.numpy as jnp
from jax import lax
from jax.experimental import pallas as pl

N_E = 1024
E_DIM = 256
BETA = 0.25
B = 16
HW = 1024
T = 1024
NT = HW // T


def _vq_block(z_ref, cb_ref, zq_ref, idx_ref, part_ref):
    zb = z_ref[0]
    cb = cb_ref[...]

    zn = jnp.sum(zb * zb, axis=0, keepdims=True)
    cbn = jnp.sum(cb * cb, axis=1)
    mmT = lax.dot_general(cb, zb, (((1,), (0,)), ((), ())),
                          preferred_element_type=jnp.float32)
    d = zn + cbn[:, None] - 2.0 * mmT

    minval = jnp.min(d, axis=0, keepdims=True)
    iota_f = lax.broadcasted_iota(jnp.int32, (N_E, T), 0).astype(jnp.float32)
    idxf = jnp.min(jnp.where(d == minval, iota_f, float(N_E)), axis=0,
                   keepdims=True)
    idx_ref[0, 0] = idxf[0, :].astype(jnp.int32)

    onehot = jnp.where(iota_f == idxf, 1.0, 0.0)
    zq = lax.dot_general(cb, onehot, (((0,), (0,)), ((), ())),
                         preferred_element_type=jnp.float32)
    zq_ref[0] = zq

    diff = zq - zb
    part = jnp.sum(diff * diff)
    part_ref[0, 0] = jnp.broadcast_to(part, (128,))


@functools.partial(jax.jit, static_argnames=())
def kernel(z, codebook):
    z3 = z.reshape(B, E_DIM, HW)
    grid = (B * NT,)

    zq3, idx2, parts = pl.pallas_call(
        _vq_block,
        grid=grid,
        in_specs=[
            pl.BlockSpec((1, E_DIM, T), lambda i: (i // NT, 0, i % NT)),
            pl.BlockSpec((N_E, E_DIM), lambda i: (0, 0)),
        ],
        out_specs=[
            pl.BlockSpec((1, E_DIM, T), lambda i: (i // NT, 0, i % NT)),
            pl.BlockSpec((1, 1, T), lambda i: (i, 0, 0)),
            pl.BlockSpec((1, 1, 128), lambda i: (i, 0, 0)),
        ],
        out_shape=[
            jax.ShapeDtypeStruct((B, E_DIM, HW), jnp.float32),
            jax.ShapeDtypeStruct((B * NT, 1, T), jnp.int32),
            jax.ShapeDtypeStruct((B * NT, 1, 128), jnp.float32),
        ],
    )(z3, codebook)

    z_q_out = zq3.reshape(z.shape)
    loss = (1.0 + BETA) * jnp.sum(parts[:, 0, 0]) / (B * HW * E_DIM)
    indices_out = idx2.reshape(B, 1, 32, 32)
    return (z_q_out, loss, indices_out)
